# baseline (speedup 1.0000x reference)
import functools

import jax
import jax.numpy as jnp
from jax import lax
from jax.experimental import pallas as pl
from jax.experimental.pallas import tpu as pltpu
from jax.experimental.pallas import tpu_sc as plsc

N = 10000
E = 320000
F_IN = 128
F_EDGE = 16
H = 256
HH = 128
NB = 4

NC = 2
NS = 16
CHUNK = 80

EPW = E // (NC * NS)
W_ITERS = EPW // CHUNK
NPAD = 10240
ROWS_PT = NPAD // NS
DEG_PT = NPAD // NS

CSEG = 64
EPT_SEG = E // NS
EPT_PAD = 20480
SEG_PAD = EPT_PAD - EPT_SEG
SEG_ITERS = EPT_PAD // CSEG
QD = 4
NG = SEG_ITERS // QD


@functools.lru_cache(maxsize=None)
def _mesh():
    return plsc.VectorSubcoreMesh(core_axis_name="c", subcore_axis_name="s",
                                  num_cores=NC, num_subcores=NS)


def _deg_body(dstW, zeros_hbm, ones_hbm, out0, out1, didx, ones_v, table, sem):
    c = lax.axis_index("c")
    s = lax.axis_index("s")
    pltpu.sync_copy(zeros_hbm, table.at[pl.ds(s * DEG_PT, DEG_PT)])
    pltpu.sync_copy(ones_hbm, ones_v)
    pltpu.sync_copy(dstW.at[s * NC + c], didx)
    plsc.subcore_barrier()

    def chunk(j, carry):
        pltpu.sync_copy(ones_v, table.at[didx.at[j]], add=True)
        return carry

    lax.fori_loop(0, W_ITERS, chunk, 0)
    plsc.subcore_barrier()

    @pl.when(c == 0)
    def _():
        pltpu.sync_copy(table.at[pl.ds(s * DEG_PT, DEG_PT)],
                        out0.at[pl.ds(s * DEG_PT, DEG_PT)])

    @pl.when(c == 1)
    def _():
        pltpu.sync_copy(table.at[pl.ds(s * DEG_PT, DEG_PT)],
                        out1.at[pl.ds(s * DEG_PT, DEG_PT)])


@functools.lru_cache(maxsize=None)
def _deg_kernel():
    return pl.kernel(
        _deg_body,
        out_type=(jax.ShapeDtypeStruct((NPAD,), jnp.float32),
                  jax.ShapeDtypeStruct((NPAD,), jnp.float32)),
        mesh=_mesh(),
        scratch_types=[
            pltpu.VMEM((W_ITERS, CHUNK), jnp.int32),
            pltpu.VMEM((CHUNK,), jnp.float32),
            pltpu.VMEM_SHARED((NPAD,), jnp.float32),
            pltpu.SemaphoreType.DMA,
        ],
    )


def _deg_call(*args):
    return _deg_kernel()(*args)


def _segsum_body(hp0, hp1, src_hbm, dst_hbm, zrows_hbm, out0, out1,
                 s0, s1, s2, s3, d0_, d1_, d2_, d3_,
                 r0, r1, r2, r3, table,
                 is0, is1, is2, is3, id0, id1, id2, id3,
                 g0, g1, g2, g3, ss0, ss1, ss2, ss3):
    c = lax.axis_index("c")
    s = lax.axis_index("s")
    pltpu.sync_copy(zrows_hbm, table.at[pl.ds(s * ROWS_PT, ROWS_PT)])
    plsc.subcore_barrier()
    base = s * EPT_PAD
    sidx = (s0, s1, s2, s3)
    didx = (d0_, d1_, d2_, d3_)
    rows = (r0, r1, r2, r3)
    isems = (is0, is1, is2, is3)
    idsems = (id0, id1, id2, id3)
    gsems = (g0, g1, g2, g3)
    ssems = (ss0, ss1, ss2, ss3)

    for b in range(QD):
        off = base + b * CSEG
        pltpu.async_copy(src_hbm.at[pl.ds(off, CSEG)], sidx[b], isems[b])
        pltpu.async_copy(dst_hbm.at[pl.ds(off, CSEG)], didx[b], idsems[b])

    def run(h_in):
        def group(i, carry):
            gd = []
            for b in range(QD):
                @pl.when(i > 0)
                def _():
                    pltpu.make_async_copy(hp0.at[pl.ds(0, CSEG)], rows[b],
                                          ssems[b]).wait()
                    off = base + (i * QD + b) * CSEG
                    pltpu.async_copy(dst_hbm.at[pl.ds(off, CSEG)], didx[b],
                                     idsems[b])
                pltpu.make_async_copy(src_hbm.at[pl.ds(0, CSEG)], sidx[b],
                                      isems[b]).wait()
                gd.append(pltpu.async_copy(h_in.at[sidx[b]], rows[b], gsems[b]))
            for b in range(QD):
                gd[b].wait()
                pltpu.make_async_copy(dst_hbm.at[pl.ds(0, CSEG)], didx[b],
                                      idsems[b]).wait()
                pltpu.async_copy(rows[b], table.at[didx[b]], add=True,
                                 sem=ssems[b])

                @pl.when(i + 1 < NG)
                def _():
                    off = base + ((i + 1) * QD + b) * CSEG
                    pltpu.async_copy(src_hbm.at[pl.ds(off, CSEG)], sidx[b],
                                     isems[b])
            return carry

        lax.fori_loop(0, NG, group, 0)
        for b in range(QD):
            pltpu.make_async_copy(hp0.at[pl.ds(0, CSEG)], rows[b],
                                  ssems[b]).wait()

    @pl.when(c == 0)
    def _():
        run(hp0)

    @pl.when(c == 1)
    def _():
        run(hp1)

    plsc.subcore_barrier()

    @pl.when(c == 0)
    def _():
        pltpu.sync_copy(table.at[pl.ds(s * ROWS_PT, ROWS_PT)],
                        out0.at[pl.ds(s * ROWS_PT, ROWS_PT)])

    @pl.when(c == 1)
    def _():
        pltpu.sync_copy(table.at[pl.ds(s * ROWS_PT, ROWS_PT)],
                        out1.at[pl.ds(s * ROWS_PT, ROWS_PT)])


@functools.lru_cache(maxsize=None)
def _segsum_kernel():
    return pl.kernel(
        _segsum_body,
        out_type=(jax.ShapeDtypeStruct((NPAD, HH), jnp.float32),
                  jax.ShapeDtypeStruct((NPAD, HH), jnp.float32)),
        mesh=_mesh(),
        scratch_types=(
            [pltpu.VMEM((CSEG,), jnp.int32)] * (2 * QD)
            + [pltpu.VMEM((CSEG, HH), jnp.float32)] * QD
            + [pltpu.VMEM_SHARED((NPAD, HH), jnp.float32)]
            + [pltpu.SemaphoreType.DMA] * (4 * QD)),
    )


def _segsum_call(*args):
    return _segsum_kernel()(*args)


def _cls_body(xs_t, xt_t, srcW, dstW, sf_out, tf_out,
              sidx, didx, a0, a1, b0, b1,
              sga0, sga1, sgb0, sgb1, swa0, swa1, swb0, swb1):
    c = lax.axis_index("c")
    s = lax.axis_index("s")
    w = s * NC + c
    base = w * EPW
    pltpu.sync_copy(srcW.at[w], sidx)
    pltpu.sync_copy(dstW.at[w], didx)
    abuf = (a0, a1)
    bbuf = (b0, b1)
    sga = (sga0, sga1)
    sgb = (sgb0, sgb1)
    swa = (swa0, swa1)
    swb = (swb0, swb1)

    def pair(i, carry):
        j0 = 2 * i
        @pl.when(i > 0)
        def _():
            for b in range(2):
                pltpu.make_async_copy(abuf[b], sf_out.at[pl.ds(0, CHUNK)],
                                      swa[b]).wait()
                pltpu.make_async_copy(bbuf[b], tf_out.at[pl.ds(0, CHUNK)],
                                      swb[b]).wait()
        gs = []
        for b in range(2):
            gs.append(pltpu.async_copy(xs_t.at[sidx.at[j0 + b]], abuf[b], sga[b]))
            gs.append(pltpu.async_copy(xt_t.at[didx.at[j0 + b]], bbuf[b], sgb[b]))
        for b in range(2):
            off = base + (j0 + b) * CHUNK
            gs[2 * b].wait()
            pltpu.async_copy(abuf[b], sf_out.at[pl.ds(off, CHUNK)], swa[b])
            gs[2 * b + 1].wait()
            pltpu.async_copy(bbuf[b], tf_out.at[pl.ds(off, CHUNK)], swb[b])
        return carry

    lax.fori_loop(0, W_ITERS // 2, pair, 0)

    for b in range(2):
        pltpu.make_async_copy(abuf[b], sf_out.at[pl.ds(0, CHUNK)], swa[b]).wait()
        pltpu.make_async_copy(bbuf[b], tf_out.at[pl.ds(0, CHUNK)], swb[b]).wait()

    j = W_ITERS - 1
    off = base + j * CHUNK
    pltpu.async_copy(xs_t.at[sidx.at[j]], a0, sga0).wait()
    pltpu.async_copy(xt_t.at[didx.at[j]], b0, sgb0).wait()
    wa = pltpu.async_copy(a0, sf_out.at[pl.ds(off, CHUNK)], swa0)
    wb = pltpu.async_copy(b0, tf_out.at[pl.ds(off, CHUNK)], swb0)
    wa.wait()
    wb.wait()


@functools.lru_cache(maxsize=None)
def _cls_kernel():
    return pl.kernel(
        _cls_body,
        out_type=(jax.ShapeDtypeStruct((E, HH), jnp.int32),
                  jax.ShapeDtypeStruct((E, HH), jnp.int32)),
        mesh=_mesh(),
        scratch_types=(
            [pltpu.VMEM((W_ITERS, CHUNK), jnp.int32)] * 2
            + [pltpu.VMEM((CHUNK, HH), jnp.int32)] * 4
            + [pltpu.SemaphoreType.DMA] * 8),
    )


def _cls_call(*args):
    return _cls_kernel()(*args)


RN = 2000


def _entry_body(nf, We, be, o):
    o[...] = jnp.maximum(
        jnp.dot(nf[...], We[...], preferred_element_type=jnp.float32) + be[...],
        0.0)


def _k1_body(x, inv, Wg, bg, hp0, hp1, base_o):
    xv = x[...]
    h = jnp.dot(xv, Wg[...], preferred_element_type=jnp.float32) + bg[...]
    iv = inv[...]
    hp = h * iv
    hp0[...] = hp[:, :HH]
    hp1[...] = hp[:, HH:]
    base_o[...] = h * (iv * iv) + xv


def _ln(t, g, b):
    m = jnp.mean(t, axis=1, keepdims=True)
    d = t - m
    v = jnp.mean(d * d, axis=1, keepdims=True)
    return d * lax.rsqrt(v + 1e-5) * g + b


def _k2_body(a0, a1, base, inv, g, b, xo):
    agg0 = jnp.concatenate([a0[...], a1[...]], axis=1)
    t = agg0 * inv[...] + base[...]
    xo[...] = _ln(t, g[...], b[...])


def _rne_bf16_bits(y):
    by = lax.bitcast_convert_type(y, jnp.int32)
    return by + 0x7FFF + (lax.shift_right_logical(by, 16) & 1)


def _pack_halves(y):
    lo = lax.shift_right_logical(_rne_bf16_bits(y[:, :HH]), 16)
    hi = (_rne_bf16_bits(y[:, HH:]) >> 16) << 16
    return hi | lo


def _k2f_body(a0, a1, base, inv, g, b, Wps, Wpt, xs_o, xt_o):
    agg0 = jnp.concatenate([a0[...], a1[...]], axis=1)
    t = agg0 * inv[...] + base[...]
    x = _ln(t, g[...], b[...])
    xs_o[...] = _pack_halves(
        jnp.dot(x, Wps[...], preferred_element_type=jnp.float32))
    xt_o[...] = _pack_halves(
        jnp.dot(x, Wpt[...], preferred_element_type=jnp.float32))


BE = 1280
BF = jnp.bfloat16


def _unpack_pair(w):
    lo = lax.bitcast_convert_type(w << 16, jnp.float32)
    hi = lax.bitcast_convert_type((w >> 16) << 16, jnp.float32)
    return lo, hi


def _mlp_body(sf, tf, ea, Wpe, bp, cW1, cb1, cW2, cb2, cg, cbb, Wo, bo, o):
    slo, shi = _unpack_pair(sf[...])
    tlo, thi = _unpack_pair(tf[...])
    z = jnp.concatenate([slo + tlo, shi + thi], axis=1) + bp[...]
    z = z + jnp.dot(ea[...], Wpe[...], preferred_element_type=jnp.float32)
    xc = jnp.maximum(z, 0.0)
    w1 = cW1[...]
    b1 = cb1[...]
    w2 = cW2[...]
    b2 = cb2[...]
    g = cg[...]
    bb = cbb[...]
    for i in range(2):
        h = jnp.maximum(
            jnp.dot(xc.astype(BF), w1[i], preferred_element_type=jnp.float32)
            + b1[i:i + 1, :], 0.0)
        h = (jnp.dot(h.astype(BF), w2[i], preferred_element_type=jnp.float32)
             + b2[i:i + 1, :])
        xc = _ln(xc + h, g[i:i + 1, :], bb[i:i + 1, :])
    o[...] = jnp.dot(xc, Wo[...], preferred_element_type=jnp.float32) + bo[...]


def _row_spec(r, cdim):
    return pl.BlockSpec((r, cdim), lambda i: (i, 0))


def _full_spec(shape):
    nd = len(shape)
    return pl.BlockSpec(shape, lambda i: (0,) * nd)


def kernel(node_features, edge_index, edge_attr, We, be, Wg, bg, lng, lnb,
           Wp, bp, cW1, cb1, cW2, cb2, cg, cbb, Wo, bo):
    f32 = jnp.float32
    src = edge_index[0].astype(jnp.int32)
    dst = edge_index[1].astype(jnp.int32)

    zeros_deg = jnp.zeros((DEG_PT,), f32)
    ones_c = jnp.ones((CHUNK,), f32)
    zrows = jnp.zeros((ROWS_PT, HH), f32)

    srcW = src.reshape(NC * NS, W_ITERS, CHUNK)
    dstW = dst.reshape(NC * NS, W_ITERS, CHUNK)

    pad_iota = jnp.arange(SEG_PAD, dtype=jnp.int32) % 128
    src_p = jnp.concatenate(
        [src.reshape(NS, EPT_SEG),
         jnp.broadcast_to(pad_iota, (NS, SEG_PAD))], axis=1).reshape(-1)
    dst_p = jnp.concatenate(
        [dst.reshape(NS, EPT_SEG),
         jnp.broadcast_to(N + pad_iota, (NS, SEG_PAD))], axis=1).reshape(-1)

    d0, d1 = _deg_call(dstW, zeros_deg, ones_c)
    inv = lax.rsqrt(d0[:N] + d1[:N] + 1.0)[:, None]

    x = pl.pallas_call(
        _entry_body,
        grid=(N // RN,),
        in_specs=[_row_spec(RN, F_IN), _full_spec((F_IN, H)), _full_spec((1, H))],
        out_specs=_row_spec(RN, H),
        out_shape=jax.ShapeDtypeStruct((N, H), f32),
    )(node_features, We, be[None])

    for i in range(NB):
        hp0, hp1, basei = pl.pallas_call(
            _k1_body,
            grid=(N // RN,),
            in_specs=[_row_spec(RN, H), _row_spec(RN, 1),
                      _full_spec((H, H)), _full_spec((1, H))],
            out_specs=[_row_spec(RN, HH), _row_spec(RN, HH), _row_spec(RN, H)],
            out_shape=[jax.ShapeDtypeStruct((N, HH), f32),
                       jax.ShapeDtypeStruct((N, HH), f32),
                       jax.ShapeDtypeStruct((N, H), f32)],
        )(x, inv, Wg[i], bg[i][None])

        a0, a1 = _segsum_call(hp0, hp1, src_p, dst_p, zrows)

        if i < NB - 1:
            x = pl.pallas_call(
                _k2_body,
                grid=(N // RN,),
                in_specs=[_row_spec(RN, HH), _row_spec(RN, HH),
                          _row_spec(RN, H), _row_spec(RN, 1),
                          _full_spec((1, H)), _full_spec((1, H))],
                out_specs=_row_spec(RN, H),
                out_shape=jax.ShapeDtypeStruct((N, H), f32),
            )(a0, a1, basei, inv, lng[i][None], lnb[i][None])
        else:
            xs, xt = pl.pallas_call(
                _k2f_body,
                grid=(N // RN,),
                in_specs=[_row_spec(RN, HH), _row_spec(RN, HH),
                          _row_spec(RN, H), _row_spec(RN, 1),
                          _full_spec((1, H)), _full_spec((1, H)),
                          _full_spec((H, H)), _full_spec((H, H))],
                out_specs=[_row_spec(RN, HH), _row_spec(RN, HH)],
                out_shape=[jax.ShapeDtypeStruct((N, HH), jnp.int32),
                           jax.ShapeDtypeStruct((N, HH), jnp.int32)],
            )(a0, a1, basei, inv, lng[i][None], lnb[i][None],
              Wp[:H], Wp[H:2 * H])

    sf, tf = _cls_call(xs, xt, srcW, dstW)

    logits = pl.pallas_call(
        _mlp_body,
        grid=(E // BE,),
        in_specs=[_row_spec(BE, HH), _row_spec(BE, HH), _row_spec(BE, F_EDGE),
                  _full_spec((F_EDGE, H)), _full_spec((1, H)),
                  _full_spec((2, H, H)), _full_spec((2, H)),
                  _full_spec((2, H, H)), _full_spec((2, H)),
                  _full_spec((2, H)), _full_spec((2, H)),
                  _full_spec((H, 2)), _full_spec((1, 2))],
        out_specs=_row_spec(BE, 2),
        out_shape=jax.ShapeDtypeStruct((E, 2), f32),
    )(sf, tf, edge_attr, Wp[2 * H:], bp[None], cW1.astype(BF), cb1,
      cW2.astype(BF), cb2, cg, cbb, Wo, bo[None])

    return logits


# speedup vs baseline: 1.0569x; 1.0569x over previous
import functools

import jax
import jax.numpy as jnp
from jax import lax
from jax.experimental import pallas as pl
from jax.experimental.pallas import tpu as pltpu
from jax.experimental.pallas import tpu_sc as plsc

N = 10000
E = 320000
F_IN = 128
F_EDGE = 16
H = 256
HH = 128
NB = 4

NC = 2
NS = 16
CHUNK = 80

EPW = E // (NC * NS)
W_ITERS = EPW // CHUNK
NPAD = 10240
ROWS_PT = NPAD // NS
DEG_PT = NPAD // NS

CSEG = 80
EPT_SEG = E // NS
EPT_PAD = 20480
SEG_PAD = EPT_PAD - EPT_SEG
SEG_ITERS = EPT_PAD // CSEG
QD = 4
NG = SEG_ITERS // QD


@functools.lru_cache(maxsize=None)
def _mesh():
    return plsc.VectorSubcoreMesh(core_axis_name="c", subcore_axis_name="s",
                                  num_cores=NC, num_subcores=NS)


def _deg_body(dstW, zeros_hbm, ones_hbm, out0, out1, didx, ones_v, table, sem):
    c = lax.axis_index("c")
    s = lax.axis_index("s")
    pltpu.sync_copy(zeros_hbm, table.at[pl.ds(s * DEG_PT, DEG_PT)])
    pltpu.sync_copy(ones_hbm, ones_v)
    pltpu.sync_copy(dstW.at[s * NC + c], didx)
    plsc.subcore_barrier()

    def chunk(j, carry):
        pltpu.sync_copy(ones_v, table.at[didx.at[j]], add=True)
        return carry

    lax.fori_loop(0, W_ITERS, chunk, 0)
    plsc.subcore_barrier()

    @pl.when(c == 0)
    def _():
        pltpu.sync_copy(table.at[pl.ds(s * DEG_PT, DEG_PT)],
                        out0.at[pl.ds(s * DEG_PT, DEG_PT)])

    @pl.when(c == 1)
    def _():
        pltpu.sync_copy(table.at[pl.ds(s * DEG_PT, DEG_PT)],
                        out1.at[pl.ds(s * DEG_PT, DEG_PT)])


@functools.lru_cache(maxsize=None)
def _deg_kernel():
    return pl.kernel(
        _deg_body,
        out_type=(jax.ShapeDtypeStruct((NPAD,), jnp.float32),
                  jax.ShapeDtypeStruct((NPAD,), jnp.float32)),
        mesh=_mesh(),
        scratch_types=[
            pltpu.VMEM((W_ITERS, CHUNK), jnp.int32),
            pltpu.VMEM((CHUNK,), jnp.float32),
            pltpu.VMEM_SHARED((NPAD,), jnp.float32),
            pltpu.SemaphoreType.DMA,
        ],
    )


def _deg_call(*args):
    return _deg_kernel()(*args)


def _segsum_body(hp0, hp1, src_hbm, dst_hbm, zrows_hbm, out0, out1,
                 s0, s1, s2, s3, d0_, d1_, d2_, d3_,
                 r0, r1, r2, r3, table,
                 is0, is1, is2, is3, id0, id1, id2, id3,
                 g0, g1, g2, g3, ss0, ss1, ss2, ss3):
    c = lax.axis_index("c")
    s = lax.axis_index("s")
    pltpu.sync_copy(zrows_hbm, table.at[pl.ds(s * ROWS_PT, ROWS_PT)])
    plsc.subcore_barrier()
    base = s * EPT_PAD
    sidx = (s0, s1, s2, s3)
    didx = (d0_, d1_, d2_, d3_)
    rows = (r0, r1, r2, r3)
    isems = (is0, is1, is2, is3)
    idsems = (id0, id1, id2, id3)
    gsems = (g0, g1, g2, g3)
    ssems = (ss0, ss1, ss2, ss3)

    for b in range(QD):
        off = base + b * CSEG
        pltpu.async_copy(src_hbm.at[pl.ds(off, CSEG)], sidx[b], isems[b])
        pltpu.async_copy(dst_hbm.at[pl.ds(off, CSEG)], didx[b], idsems[b])

    def run(h_in):
        def group(i, carry):
            gd = []
            for b in range(QD):
                @pl.when(i > 0)
                def _():
                    pltpu.make_async_copy(hp0.at[pl.ds(0, CSEG)], rows[b],
                                          ssems[b]).wait()
                    off = base + (i * QD + b) * CSEG
                    pltpu.async_copy(dst_hbm.at[pl.ds(off, CSEG)], didx[b],
                                     idsems[b])
                pltpu.make_async_copy(src_hbm.at[pl.ds(0, CSEG)], sidx[b],
                                      isems[b]).wait()
                gd.append(pltpu.async_copy(h_in.at[sidx[b]], rows[b], gsems[b]))
            for b in range(QD):
                gd[b].wait()
                pltpu.make_async_copy(dst_hbm.at[pl.ds(0, CSEG)], didx[b],
                                      idsems[b]).wait()
                pltpu.async_copy(rows[b], table.at[didx[b]], add=True,
                                 sem=ssems[b])

                @pl.when(i + 1 < NG)
                def _():
                    off = base + ((i + 1) * QD + b) * CSEG
                    pltpu.async_copy(src_hbm.at[pl.ds(off, CSEG)], sidx[b],
                                     isems[b])
            return carry

        lax.fori_loop(0, NG, group, 0)
        for b in range(QD):
            pltpu.make_async_copy(hp0.at[pl.ds(0, CSEG)], rows[b],
                                  ssems[b]).wait()

    @pl.when(c == 0)
    def _():
        run(hp0)

    @pl.when(c == 1)
    def _():
        run(hp1)

    plsc.subcore_barrier()

    @pl.when(c == 0)
    def _():
        pltpu.sync_copy(table.at[pl.ds(s * ROWS_PT, ROWS_PT)],
                        out0.at[pl.ds(s * ROWS_PT, ROWS_PT)])

    @pl.when(c == 1)
    def _():
        pltpu.sync_copy(table.at[pl.ds(s * ROWS_PT, ROWS_PT)],
                        out1.at[pl.ds(s * ROWS_PT, ROWS_PT)])


@functools.lru_cache(maxsize=None)
def _segsum_kernel():
    return pl.kernel(
        _segsum_body,
        out_type=(jax.ShapeDtypeStruct((NPAD, HH), jnp.float32),
                  jax.ShapeDtypeStruct((NPAD, HH), jnp.float32)),
        mesh=_mesh(),
        scratch_types=(
            [pltpu.VMEM((CSEG,), jnp.int32)] * (2 * QD)
            + [pltpu.VMEM((CSEG, HH), jnp.float32)] * QD
            + [pltpu.VMEM_SHARED((NPAD, HH), jnp.float32)]
            + [pltpu.SemaphoreType.DMA] * (4 * QD)),
    )


def _segsum_call(*args):
    return _segsum_kernel()(*args)


def _cls_body(xs_t, xt_t, srcW, dstW, sf_out, tf_out,
              sidx, didx, a0, a1, b0, b1,
              sga0, sga1, sgb0, sgb1, swa0, swa1, swb0, swb1):
    c = lax.axis_index("c")
    s = lax.axis_index("s")
    w = s * NC + c
    base = w * EPW
    pltpu.sync_copy(srcW.at[w], sidx)
    pltpu.sync_copy(dstW.at[w], didx)
    abuf = (a0, a1)
    bbuf = (b0, b1)
    sga = (sga0, sga1)
    sgb = (sgb0, sgb1)
    swa = (swa0, swa1)
    swb = (swb0, swb1)

    def pair(i, carry):
        j0 = 2 * i
        @pl.when(i > 0)
        def _():
            for b in range(2):
                pltpu.make_async_copy(abuf[b], sf_out.at[pl.ds(0, CHUNK)],
                                      swa[b]).wait()
                pltpu.make_async_copy(bbuf[b], tf_out.at[pl.ds(0, CHUNK)],
                                      swb[b]).wait()
        gs = []
        for b in range(2):
            gs.append(pltpu.async_copy(xs_t.at[sidx.at[j0 + b]], abuf[b], sga[b]))
            gs.append(pltpu.async_copy(xt_t.at[didx.at[j0 + b]], bbuf[b], sgb[b]))
        for b in range(2):
            off = base + (j0 + b) * CHUNK
            gs[2 * b].wait()
            pltpu.async_copy(abuf[b], sf_out.at[pl.ds(off, CHUNK)], swa[b])
            gs[2 * b + 1].wait()
            pltpu.async_copy(bbuf[b], tf_out.at[pl.ds(off, CHUNK)], swb[b])
        return carry

    lax.fori_loop(0, W_ITERS // 2, pair, 0)

    for b in range(2):
        pltpu.make_async_copy(abuf[b], sf_out.at[pl.ds(0, CHUNK)], swa[b]).wait()
        pltpu.make_async_copy(bbuf[b], tf_out.at[pl.ds(0, CHUNK)], swb[b]).wait()

    j = W_ITERS - 1
    off = base + j * CHUNK
    pltpu.async_copy(xs_t.at[sidx.at[j]], a0, sga0).wait()
    pltpu.async_copy(xt_t.at[didx.at[j]], b0, sgb0).wait()
    wa = pltpu.async_copy(a0, sf_out.at[pl.ds(off, CHUNK)], swa0)
    wb = pltpu.async_copy(b0, tf_out.at[pl.ds(off, CHUNK)], swb0)
    wa.wait()
    wb.wait()


@functools.lru_cache(maxsize=None)
def _cls_kernel():
    return pl.kernel(
        _cls_body,
        out_type=(jax.ShapeDtypeStruct((E, HH), jnp.int32),
                  jax.ShapeDtypeStruct((E, HH), jnp.int32)),
        mesh=_mesh(),
        scratch_types=(
            [pltpu.VMEM((W_ITERS, CHUNK), jnp.int32)] * 2
            + [pltpu.VMEM((CHUNK, HH), jnp.int32)] * 4
            + [pltpu.SemaphoreType.DMA] * 8),
    )


def _cls_call(*args):
    return _cls_kernel()(*args)


RN = 2000


def _emit_k1(xv, iv, Wg_ref, bg_ref, hp0, hp1, base_o):
    h = jnp.dot(xv, Wg_ref[...], preferred_element_type=jnp.float32) + bg_ref[...]
    hp = h * iv
    hp0[...] = hp[:, :HH]
    hp1[...] = hp[:, HH:]
    base_o[...] = h * (iv * iv) + xv


def _entry_k1_body(nf, inv, We, be, Wg, bg, hp0, hp1, base_o):
    xv = jnp.maximum(
        jnp.dot(nf[...], We[...], preferred_element_type=jnp.float32) + be[...],
        0.0)
    _emit_k1(xv, inv[...], Wg, bg, hp0, hp1, base_o)


def _k21_body(a0, a1, base, inv, g, b, Wg, bg, hp0, hp1, base_o):
    agg0 = jnp.concatenate([a0[...], a1[...]], axis=1)
    iv = inv[...]
    t = agg0 * iv + base[...]
    xv = _ln(t, g[...], b[...])
    _emit_k1(xv, iv, Wg, bg, hp0, hp1, base_o)


def _ln(t, g, b):
    m = jnp.mean(t, axis=1, keepdims=True)
    d = t - m
    v = jnp.mean(d * d, axis=1, keepdims=True)
    return d * lax.rsqrt(v + 1e-5) * g + b


def _rne_bf16_bits(y):
    by = lax.bitcast_convert_type(y, jnp.int32)
    return by + 0x7FFF + (lax.shift_right_logical(by, 16) & 1)


def _pack_halves(y):
    lo = lax.shift_right_logical(_rne_bf16_bits(y[:, :HH]), 16)
    hi = (_rne_bf16_bits(y[:, HH:]) >> 16) << 16
    return hi | lo


def _k2f_body(a0, a1, base, inv, g, b, Wps, Wpt, xs_o, xt_o):
    agg0 = jnp.concatenate([a0[...], a1[...]], axis=1)
    t = agg0 * inv[...] + base[...]
    x = _ln(t, g[...], b[...])
    xs_o[...] = _pack_halves(
        jnp.dot(x, Wps[...], preferred_element_type=jnp.float32))
    xt_o[...] = _pack_halves(
        jnp.dot(x, Wpt[...], preferred_element_type=jnp.float32))


BE = 2560
BF = jnp.bfloat16


def _unpack_pair(w):
    lo = lax.bitcast_convert_type(w << 16, jnp.float32)
    hi = lax.bitcast_convert_type((w >> 16) << 16, jnp.float32)
    return lo, hi


def _mlp_body(sf, tf, ea, Wpe, bp, cW1, cb1, cW2, cb2, cg, cbb, Wo, bo, o):
    slo, shi = _unpack_pair(sf[...])
    tlo, thi = _unpack_pair(tf[...])
    z = jnp.concatenate([slo + tlo, shi + thi], axis=1) + bp[...]
    z = z + jnp.dot(ea[...], Wpe[...], preferred_element_type=jnp.float32)
    xc = jnp.maximum(z, 0.0)
    w1 = cW1[...]
    b1 = cb1[...]
    w2 = cW2[...]
    b2 = cb2[...]
    g = cg[...]
    bb = cbb[...]
    for i in range(2):
        h = jnp.maximum(
            jnp.dot(xc.astype(BF), w1[i], preferred_element_type=jnp.float32)
            + b1[i:i + 1, :], 0.0)
        h = (jnp.dot(h.astype(BF), w2[i], preferred_element_type=jnp.float32)
             + b2[i:i + 1, :])
        xc = _ln(xc + h, g[i:i + 1, :], bb[i:i + 1, :])
    o[...] = jnp.dot(xc, Wo[...], preferred_element_type=jnp.float32) + bo[...]


def _row_spec(r, cdim):
    return pl.BlockSpec((r, cdim), lambda i: (i, 0))


def _full_spec(shape):
    nd = len(shape)
    return pl.BlockSpec(shape, lambda i: (0,) * nd)


def kernel(node_features, edge_index, edge_attr, We, be, Wg, bg, lng, lnb,
           Wp, bp, cW1, cb1, cW2, cb2, cg, cbb, Wo, bo):
    f32 = jnp.float32
    src = edge_index[0].astype(jnp.int32)
    dst = edge_index[1].astype(jnp.int32)

    zeros_deg = jnp.zeros((DEG_PT,), f32)
    ones_c = jnp.ones((CHUNK,), f32)
    zrows = jnp.zeros((ROWS_PT, HH), f32)

    srcW = src.reshape(NC * NS, W_ITERS, CHUNK)
    dstW = dst.reshape(NC * NS, W_ITERS, CHUNK)

    pad_iota = jnp.arange(SEG_PAD, dtype=jnp.int32) % 128
    src_p = jnp.concatenate(
        [src.reshape(NS, EPT_SEG),
         jnp.broadcast_to(pad_iota, (NS, SEG_PAD))], axis=1).reshape(-1)
    dst_p = jnp.concatenate(
        [dst.reshape(NS, EPT_SEG),
         jnp.broadcast_to(N + pad_iota, (NS, SEG_PAD))], axis=1).reshape(-1)

    d0, d1 = _deg_call(dstW, zeros_deg, ones_c)
    inv = lax.rsqrt(d0[:N] + d1[:N] + 1.0)[:, None]

    k1_out_specs = [_row_spec(RN, HH), _row_spec(RN, HH), _row_spec(RN, H)]
    k1_out_shape = [jax.ShapeDtypeStruct((N, HH), f32),
                    jax.ShapeDtypeStruct((N, HH), f32),
                    jax.ShapeDtypeStruct((N, H), f32)]
    hp0, hp1, basei = pl.pallas_call(
        _entry_k1_body,
        grid=(N // RN,),
        in_specs=[_row_spec(RN, F_IN), _row_spec(RN, 1),
                  _full_spec((F_IN, H)), _full_spec((1, H)),
                  _full_spec((H, H)), _full_spec((1, H))],
        out_specs=k1_out_specs,
        out_shape=k1_out_shape,
    )(node_features, inv, We, be[None], Wg[0], bg[0][None])

    for i in range(NB):
        a0, a1 = _segsum_call(hp0, hp1, src_p, dst_p, zrows)
        if i < NB - 1:
            hp0, hp1, basei = pl.pallas_call(
                _k21_body,
                grid=(N // RN,),
                in_specs=[_row_spec(RN, HH), _row_spec(RN, HH),
                          _row_spec(RN, H), _row_spec(RN, 1),
                          _full_spec((1, H)), _full_spec((1, H)),
                          _full_spec((H, H)), _full_spec((1, H))],
                out_specs=k1_out_specs,
                out_shape=k1_out_shape,
            )(a0, a1, basei, inv, lng[i][None], lnb[i][None],
              Wg[i + 1], bg[i + 1][None])
        else:
            xs, xt = pl.pallas_call(
                _k2f_body,
                grid=(N // RN,),
                in_specs=[_row_spec(RN, HH), _row_spec(RN, HH),
                          _row_spec(RN, H), _row_spec(RN, 1),
                          _full_spec((1, H)), _full_spec((1, H)),
                          _full_spec((H, H)), _full_spec((H, H))],
                out_specs=[_row_spec(RN, HH), _row_spec(RN, HH)],
                out_shape=[jax.ShapeDtypeStruct((N, HH), jnp.int32),
                           jax.ShapeDtypeStruct((N, HH), jnp.int32)],
            )(a0, a1, basei, inv, lng[i][None], lnb[i][None],
              Wp[:H], Wp[H:2 * H])

    sf, tf = _cls_call(xs, xt, srcW, dstW)

    logits = pl.pallas_call(
        _mlp_body,
        grid=(E // BE,),
        in_specs=[_row_spec(BE, HH), _row_spec(BE, HH), _row_spec(BE, F_EDGE),
                  _full_spec((F_EDGE, H)), _full_spec((1, H)),
                  _full_spec((2, H, H)), _full_spec((2, H)),
                  _full_spec((2, H, H)), _full_spec((2, H)),
                  _full_spec((2, H)), _full_spec((2, H)),
                  _full_spec((H, 2)), _full_spec((1, 2))],
        out_specs=_row_spec(BE, 2),
        out_shape=jax.ShapeDtypeStruct((E, 2), f32),
    )(sf, tf, edge_attr, Wp[2 * H:], bp[None], cW1.astype(BF), cb1,
      cW2.astype(BF), cb2, cg, cbb, Wo, bo[None])

    return logits
